# Initial kernel scaffold; baseline (speedup 1.0000x reference)
import jax
import jax.numpy as jnp
from jax.experimental import pallas as pl


def kernel(added_points, original_points, boxes):
    raise NotImplementedError("write your pallas kernel here")


# speedup vs baseline: 1.1755x; 1.1755x over previous
import functools

import jax
import jax.numpy as jnp
from jax.experimental import pallas as pl
from jax.experimental.pallas import tpu as pltpu

_GRID = 256
_VOX = 0.8
_BH = 32


def _loss_kernel(boxes_ref, added_ref, orig_ref, out_ref, acc_ref, *, nh, inv_b):
    b = pl.program_id(0)
    h = pl.program_id(1)

    @pl.when(jnp.logical_and(b == 0, h == 0))
    def _init_out():
        out_ref[...] = jnp.zeros((1, 1), jnp.float32)

    @pl.when(h == 0)
    def _init_acc():
        acc_ref[0] = 0.0
        acc_ref[1] = 0.0

    pred = jnp.sum(added_ref[0], axis=0)
    orig = jnp.sum(orig_ref[0, 1:], axis=0)

    row = (jax.lax.broadcasted_iota(jnp.int32, (_BH, _GRID), 0) + h * _BH).astype(jnp.float32)
    col = jax.lax.broadcasted_iota(jnp.int32, (_BH, _GRID), 1).astype(jnp.float32)
    x = (row - _GRID / 2.0) * _VOX
    y = (col - _GRID / 2.0) * _VOX

    bx = boxes_ref[0]
    cx = bx[:, 0][:, None, None]
    cy = bx[:, 1][:, None, None]
    cz = bx[:, 2][:, None, None]
    dx = bx[:, 3][:, None, None]
    dy = bx[:, 4][:, None, None]
    dz = bx[:, 5][:, None, None]
    heading = bx[:, 6][:, None, None]
    c = jnp.cos(-heading)
    s = jnp.sin(-heading)
    sx = x[None, :, :] - cx
    sy = y[None, :, :] - cy
    sz = _VOX - cz
    lx = sx * c - sy * s
    ly = sx * s + sy * c
    in_box = (
        (jnp.abs(lx) <= dx * 0.5)
        & (jnp.abs(ly) <= dy * 0.5)
        & (jnp.abs(sz) <= dz * 0.5)
    )
    mask = jnp.any(in_box, axis=0)

    p = (pred != 0.0) & mask
    o = (orig != 0.0) & mask
    inter = jnp.sum((p & o).astype(jnp.float32))
    union = jnp.sum((p | o).astype(jnp.float32))
    acc_ref[0] += inter
    acc_ref[1] += union

    @pl.when(h == nh - 1)
    def _finish():
        iou = acc_ref[0] / jnp.maximum(acc_ref[1], 1.0)
        out_ref[...] += jnp.full((1, 1), iou * inv_b, jnp.float32)


def kernel(added_points, original_points, boxes):
    bsz, chans, g, _ = added_points.shape
    chans_o = original_points.shape[1]
    t = boxes.shape[1]
    nh = g // _BH

    out = pl.pallas_call(
        functools.partial(_loss_kernel, nh=nh, inv_b=1.0 / bsz),
        grid=(bsz, nh),
        in_specs=[
            pl.BlockSpec((1, t, 7), lambda b, h: (b, 0, 0)),
            pl.BlockSpec((1, chans, _BH, g), lambda b, h: (b, 0, h, 0)),
            pl.BlockSpec((1, chans_o, _BH, g), lambda b, h: (b, 0, h, 0)),
        ],
        out_specs=pl.BlockSpec((1, 1), lambda b, h: (0, 0)),
        out_shape=jax.ShapeDtypeStruct((1, 1), jnp.float32),
        scratch_shapes=[pltpu.SMEM((2,), jnp.float32)],
        compiler_params=pltpu.CompilerParams(
            dimension_semantics=("arbitrary", "arbitrary"),
        ),
    )(boxes, added_points, original_points)
    return out[0, 0]
